# baseline (speedup 1.0000x reference)
import functools

import jax
import jax.numpy as jnp
from jax import lax
from jax.experimental import pallas as pl
from jax.experimental.pallas import tpu as pltpu
from jax.experimental.pallas import tpu_sc as plsc

VOC = 1000000
D = 64
NB, NP = 4096, 200
NC, NS, L = 2, 16, 16
NW = NC * NS
BLK = 128
PITCH = 129
NFULL = VOC // BLK
TAIL0 = NFULL * BLK
SLOTS1 = 246
SLOTS2 = 201

_mesh = dict(core_axis_name="c", subcore_axis_name="s")


def _wid():
    return lax.axis_index("s") * NC + lax.axis_index("c")


def _repack_body(tt_hbm, scr_hbm, slab0, slab1, dstp0, dstp1, tail_v,
                 rsem0, rsem1, wsem0, wsem1):
    w = _wid()
    slabs = (slab0, slab1)
    dsts = (dstp0, dstp1)
    rsems = (rsem0, rsem1)
    wsems = (wsem0, wsem1)
    i0s = [lax.iota(jnp.int32, L) + m * L for m in range(BLK // L)]

    def blk_of(s):
        b = w + NW * s
        return jnp.where(b < NFULL, b, w)

    def fire_read(s, b):
        off = pl.multiple_of(blk_of(s) * BLK, BLK)
        pltpu.async_copy(tt_hbm.at[:, pl.ds(off, BLK)], slabs[b], rsems[b])

    fire_read(0, 0)
    fire_read(1, 1)

    @pl.loop(0, SLOTS1, step=2)
    def _outer(g):
        for b in range(2):
            s = g + b
            pltpu.make_async_copy(
                tt_hbm.at[:, pl.ds(0, BLK)], slabs[b], rsems[b]).wait()

            @pl.when(s >= 2)
            def _drain_w():
                pltpu.make_async_copy(
                    dsts[b].at[:, pl.ds(0, BLK)],
                    scr_hbm.at[pl.ds(0, BLK), :], wsems[b]).wait()

            slab = slabs[b]
            dst = dsts[b]

            @plsc.parallel_loop(0, D, step=1, unroll=4)
            def _t(j):
                jv = jnp.full((L,), j, jnp.int32)
                for m in range(BLK // L):
                    vals = slab[j, pl.ds(m * L, L)]
                    plsc.store_scatter(dst, [i0s[m], jv], vals * 8.0)

            roff = pl.multiple_of(blk_of(s) * BLK, BLK)
            pltpu.async_copy(dst.at[:, pl.ds(0, BLK)],
                             scr_hbm.at[pl.ds(roff, BLK), :], wsems[b])

            @pl.when(s + 2 < SLOTS1)
            def _next():
                fire_read(s + 2, b)

    for b in range(2):
        pltpu.make_async_copy(
            dsts[b].at[:, pl.ds(0, BLK)],
            scr_hbm.at[pl.ds(0, BLK), :], wsems[b]).wait()

    @pl.when(w == NW - 1)
    def _tail():
        ntail = VOC - TAIL0
        for j in range(D):
            pltpu.sync_copy(tt_hbm.at[j, pl.ds(TAIL0, ntail)],
                            tail_v.at[j])

        @plsc.parallel_loop(0, D, step=1, unroll=4)
        def _tt(j):
            jv = jnp.full((L,), j, jnp.int32)
            for m in range(ntail // L):
                vals = tail_v[j, pl.ds(m * L, L)]
                plsc.store_scatter(dstp0, [i0s[m], jv], vals * 8.0)

        pltpu.sync_copy(dstp0.at[pl.ds(0, ntail), pl.ds(0, BLK)],
                        scr_hbm.at[pl.ds(TAIL0, ntail), :])


def _gather_body(scr_hbm, xt_hbm, out_hbm, idx_v, r0, r1, r2v,
                 d0, d1, gsem0, gsem1, gsem2, osem0, osem1):
    w = _wid()
    rows = (r0, r1, r2v)
    gsems = (gsem0, gsem1, gsem2)
    dsts = (d0, d1)
    osems = (osem0, osem1)
    i0s = [lax.iota(jnp.int32, L) + k * L for k in range(D // L)]

    woff = pl.multiple_of(w * 128, 128)
    pltpu.sync_copy(xt_hbm.at[:, pl.ds(woff, 128)], idx_v)

    def p_of(s):
        return jnp.where(s < NP, s, NP - 1)

    def fire_gather(s, b):
        pltpu.async_copy(scr_hbm.at[idx_v.at[p_of(s)]], rows[b], gsems[b])

    fire_gather(0, 0)
    fire_gather(1, 1)
    fire_gather(2, 2)

    @pl.loop(0, SLOTS2, step=3)
    def _outer(g):
        for b in range(3):
            s = g + b
            p = p_of(s)
            d = b % 2
            pltpu.make_async_copy(
                scr_hbm.at[idx_v.at[0]], rows[b], gsems[b]).wait()

            @pl.when(s >= 2)
            def _drain_o():
                pltpu.make_async_copy(
                    dsts[d].at[:, pl.ds(0, 128)],
                    out_hbm.at[0, :, pl.ds(0, 128)], osems[d]).wait()

            row = rows[b]
            dst = dsts[d]

            @plsc.parallel_loop(0, 128, step=1, unroll=4)
            def _t(t):
                tv = jnp.full((L,), t, jnp.int32)
                for k in range(D // L):
                    vals = row[t, pl.ds(k * L, L)]
                    plsc.store_scatter(dst, [i0s[k], tv], vals)

            pltpu.async_copy(
                dst.at[:, pl.ds(0, 128)],
                out_hbm.at[p, :, pl.ds(woff, 128)], osems[d])

            @pl.when(s + 3 < SLOTS2)
            def _next():
                fire_gather(s + 3, b)

    for d in range(2):
        pltpu.make_async_copy(
            dsts[d].at[:, pl.ds(0, 128)],
            out_hbm.at[0, :, pl.ds(0, 128)], osems[d]).wait()


def _make_k1():
    return pl.kernel(
        _repack_body,
        out_type=jax.ShapeDtypeStruct((VOC, 128), jnp.float32),
        mesh=plsc.VectorSubcoreMesh(**_mesh),
        scratch_types=[
            pltpu.VMEM((D, BLK), jnp.float32),
            pltpu.VMEM((D, BLK), jnp.float32),
            pltpu.VMEM((BLK, PITCH), jnp.float32),
            pltpu.VMEM((BLK, PITCH), jnp.float32),
            pltpu.VMEM((D, D), jnp.float32),
            pltpu.SemaphoreType.DMA,
            pltpu.SemaphoreType.DMA,
            pltpu.SemaphoreType.DMA,
            pltpu.SemaphoreType.DMA,
        ],
        compiler_params=pltpu.CompilerParams(
            use_tc_tiling_on_sc=True, needs_layout_passes=False),
    )


def _make_k2():
    return pl.kernel(
        _gather_body,
        out_type=jax.ShapeDtypeStruct((NP, D, NB), jnp.float32),
        mesh=plsc.VectorSubcoreMesh(**_mesh),
        scratch_types=[
            pltpu.VMEM((NP, 128), jnp.int32),
            pltpu.VMEM((128, 128), jnp.float32),
            pltpu.VMEM((128, 128), jnp.float32),
            pltpu.VMEM((128, 128), jnp.float32),
            pltpu.VMEM((D, PITCH), jnp.float32),
            pltpu.VMEM((D, PITCH), jnp.float32),
            pltpu.SemaphoreType.DMA,
            pltpu.SemaphoreType.DMA,
            pltpu.SemaphoreType.DMA,
            pltpu.SemaphoreType.DMA,
            pltpu.SemaphoreType.DMA,
        ],
        compiler_params=pltpu.CompilerParams(
            use_tc_tiling_on_sc=True, needs_layout_passes=False),
    )


@jax.jit
def kernel(x, table):
    xt = x.astype(jnp.int32).T
    tt = table.T
    scr = _make_k1()(tt)
    out3 = _make_k2()(scr, xt)
    return out3.transpose(2, 0, 1)


# speedup vs baseline: 1.7637x; 1.7637x over previous
import functools
import math

import jax
import jax.numpy as jnp
from jax import lax
from jax.experimental import pallas as pl
from jax.experimental.pallas import tpu as pltpu
from jax.experimental.pallas import tpu_sc as plsc

D = 64
NC, NS, L = 2, 16, 16
NW = NC * NS
IDX_MINOR = 128
K = 4
R = K * IDX_MINOR
SCALE = math.sqrt(D)


def _make_emb(n_rows_per_w: int):
    ng = n_rows_per_w // K
    b_per_w = n_rows_per_w * IDX_MINOR
    assert n_rows_per_w % K == 0 and ng % 2 == 0

    mesh = plsc.VectorSubcoreMesh(core_axis_name="c", subcore_axis_name="s")

    def body(x_hbm, table_hbm, out_hbm, idx_v, buf0, buf1, sem0, sem1):
        wid = lax.axis_index("s") * NC + lax.axis_index("c")
        pltpu.sync_copy(x_hbm.at[wid], idx_v)
        bufs = (buf0, buf1)
        sems = (sem0, sem1)
        out_base = wid * b_per_w

        def fire(grp, b):
            for j in range(K):
                pltpu.async_copy(
                    table_hbm.at[idx_v.at[grp * K + j]],
                    bufs[b].at[pl.ds(j * IDX_MINOR, IDX_MINOR)],
                    sems[b],
                )

        fire(0, 0)
        fire(1, 1)

        @pl.loop(0, ng, step=2)
        def _outer(g):
            for b in range(2):
                grp = g + b
                buf = bufs[b]
                pltpu.make_async_copy(
                    table_hbm.at[pl.ds(0, R)], buf, sems[b]
                ).wait()

                @plsc.parallel_loop(0, R, step=1, unroll=8)
                def _scale(r):
                    for c in range(D // L):
                        sl = pl.ds(c * L, L)
                        buf[r, sl] = buf[r, sl] * SCALE

                pltpu.sync_copy(
                    buf,
                    out_hbm.at[pl.ds(out_base + grp * R, R), pl.ds(0, D)],
                )

                @pl.when(grp + 2 < ng)
                def _next():
                    fire(grp + 2, b)

    kern = pl.kernel(
        body,
        out_type=jax.ShapeDtypeStruct((NW * b_per_w, 2 * D), jnp.float32),
        mesh=mesh,
        scratch_types=[
            pltpu.VMEM((n_rows_per_w, IDX_MINOR), jnp.int32),
            pltpu.VMEM((R, D), jnp.float32),
            pltpu.VMEM((R, D), jnp.float32),
            pltpu.SemaphoreType.DMA,
            pltpu.SemaphoreType.DMA,
        ],
        compiler_params=pltpu.CompilerParams(use_tc_tiling_on_sc=False),
    )
    return kern


@jax.jit
def kernel(x, table):
    n, m = x.shape
    total = n * m
    n_rows_per_w = total // (NW * IDX_MINOR)
    xr = x.astype(jnp.int32).reshape(NW, n_rows_per_w, IDX_MINOR)
    out = _make_emb(n_rows_per_w)(xr, table)
    return out[:, :D].reshape(n, m, D)
